# baseline (speedup 1.0000x reference)
import jax
import jax.numpy as jnp
from jax import lax
from jax.experimental import pallas as pl
from jax.experimental.pallas import tpu as pltpu

_EPS = 1e-5


def _im2col(xp, KH, KW, dil, Hout, Wout, Cin):
    M = Hout * Wout
    pieces = []
    for ky in range(KH):
        for kx in range(KW):
            win = xp[ky * dil:ky * dil + Hout, kx * dil:kx * dil + Wout, :]
            pieces.append(win.reshape(M, Cin))
    return pieces[0] if len(pieces) == 1 else jnp.concatenate(pieces, axis=1)


def _conv_t(xp, w_ref, KH, KW, dil, Hout, Wout, Cin):
    patches = _im2col(xp, KH, KW, dil, Hout, Wout, Cin)
    return lax.dot_general(w_ref[...], patches,
                           (((0,), (1,)), ((), ())),
                           preferred_element_type=jnp.float32)


def _make_stats_kernel(KH, KW, dil, pad, Hout, Wout, Cin):
    def _body(x_ref, w_ref, xp_ref, st_ref):
        xc = x_ref[0]
        xt = jnp.transpose(xc, (1, 2, 0)).astype(jnp.bfloat16)
        xpad = jnp.pad(xt, ((pad, pad), (pad, pad), (0, 0)))
        xp_ref[0] = xpad
        y = _conv_t(xpad, w_ref, KH, KW, dil, Hout, Wout, Cin)
        s1 = jnp.sum(y, axis=1, keepdims=True)
        s2 = jnp.sum(y * y, axis=1, keepdims=True)
        st_ref[0] = jnp.concatenate([s1, s2], axis=1)
    return _body


def _make_apply_kernel(KH, KW, dil, Hout, Wout, Cin):
    def _body(xp_ref, w_ref, sc_ref, sh_ref, o_ref):
        y = _conv_t(xp_ref[0], w_ref, KH, KW, dil, Hout, Wout, Cin)
        o_ref[0] = y * sc_ref[...] + sh_ref[...]
    return _body


def kernel(x_nchw, w_hwio, gamma, beta):
    pad, dil = 2, 2
    N, Cin, H, W = x_nchw.shape
    KH, KW, _, Cout = w_hwio.shape
    Hout = H + 2 * pad - dil * (KH - 1)
    Wout = W + 2 * pad - dil * (KW - 1)
    Hp, Wp = H + 2 * pad, W + 2 * pad
    M = Hout * Wout
    K = KH * KW * Cin

    w_flat = w_hwio.reshape(K, Cout).astype(jnp.bfloat16)

    xp, stats = pl.pallas_call(
        _make_stats_kernel(KH, KW, dil, pad, Hout, Wout, Cin),
        out_shape=(jax.ShapeDtypeStruct((N, Hp, Wp, Cin), jnp.bfloat16),
                   jax.ShapeDtypeStruct((N, Cout, 2), jnp.float32)),
        grid=(N,),
        in_specs=[
            pl.BlockSpec((1, Cin, H, W), lambda n: (n, 0, 0, 0)),
            pl.BlockSpec((K, Cout), lambda n: (0, 0)),
        ],
        out_specs=(
            pl.BlockSpec((1, Hp, Wp, Cin), lambda n: (n, 0, 0, 0)),
            pl.BlockSpec((1, Cout, 2), lambda n: (n, 0, 0)),
        ),
        compiler_params=pltpu.CompilerParams(dimension_semantics=("parallel",)),
    )(x_nchw, w_flat)

    cnt = jnp.float32(N * M)
    tot = jnp.sum(stats, axis=0)
    mean = tot[:, 0] / cnt
    var = jnp.maximum(tot[:, 1] / cnt - mean * mean, 0.0)
    scale = gamma.astype(jnp.float32) * lax.rsqrt(var + _EPS)
    shift = beta.astype(jnp.float32) - mean * scale

    out = pl.pallas_call(
        _make_apply_kernel(KH, KW, dil, Hout, Wout, Cin),
        out_shape=jax.ShapeDtypeStruct((N, Cout, M), jnp.float32),
        grid=(N,),
        in_specs=[
            pl.BlockSpec((1, Hp, Wp, Cin), lambda n: (n, 0, 0, 0)),
            pl.BlockSpec((K, Cout), lambda n: (0, 0)),
            pl.BlockSpec((Cout, 1), lambda n: (0, 0)),
            pl.BlockSpec((Cout, 1), lambda n: (0, 0)),
        ],
        out_specs=pl.BlockSpec((1, Cout, M), lambda n: (n, 0, 0)),
        compiler_params=pltpu.CompilerParams(dimension_semantics=("parallel",)),
    )(xp, w_flat, scale.reshape(Cout, 1), shift.reshape(Cout, 1))

    return out.reshape(N, Cout, Hout, Wout)
